# baseline (speedup 1.0000x reference)
import jax
import jax.numpy as jnp
from jax.experimental import pallas as pl
from jax.experimental.pallas import tpu as pltpu
from jax.experimental.pallas import tpu_sc as plsc

NUM_FEATURES = 26
VOCAB = 100000
EMBED_DIM = 16
BATCH = 16384
NUM_IDX = BATCH * NUM_FEATURES
WINDOW = 128
SUPER_ROWS = NUM_FEATURES * VOCAB // 8
ROWS_PER_F = VOCAB // 8


def _repack_tables(tables):
    tab_t = jnp.transpose(tables, (0, 2, 1)).reshape(
        NUM_FEATURES // 2, 2, EMBED_DIM, VOCAB
    )

    def body(in_ref, out_ref):
        vcc = ROWS_PER_F // 4
        for ff in range(2):
            for c in range(4):
                t = jnp.concatenate(
                    [
                        in_ref[0, ff, :, pl.ds(j * ROWS_PER_F + c * vcc, vcc)]
                        for j in range(8)
                    ],
                    axis=0,
                )
                out_ref[
                    0, pl.ds(ff * ROWS_PER_F + c * vcc, vcc), :
                ] = jnp.transpose(t)

    out = pl.pallas_call(
        body,
        grid=(NUM_FEATURES // 2,),
        in_specs=[
            pl.BlockSpec((1, 2, EMBED_DIM, VOCAB), lambda f: (f, 0, 0, 0))
        ],
        out_specs=pl.BlockSpec((1, 2 * ROWS_PER_F, 128), lambda f: (f, 0, 0)),
        out_shape=jax.ShapeDtypeStruct(
            (NUM_FEATURES // 2, 2 * ROWS_PER_F, 128), jnp.float32
        ),
        compiler_params=pltpu.CompilerParams(
            dimension_semantics=("parallel",),
            vmem_limit_bytes=64 * 1024 * 1024,
        ),
    )(tab_t)
    return out.reshape(SUPER_ROWS, 128)


def kernel(categorical_features, tables):
    flat_tables = _repack_tables(tables).reshape(NUM_FEATURES * VOCAB, EMBED_DIM)
    offs = jnp.arange(NUM_FEATURES, dtype=jnp.int32) * ROWS_PER_F
    v = jnp.transpose(categorical_features.astype(jnp.int32))
    j, vm = jnp.divmod(v, ROWS_PER_F)
    flat_idx = (8 * (offs[:, None] + vm) + j).reshape(1, NUM_IDX)

    mesh = plsc.VectorSubcoreMesh(core_axis_name="core", subcore_axis_name="subcore")

    @pl.kernel(
        out_type=jax.ShapeDtypeStruct((NUM_IDX, EMBED_DIM), jnp.float32),
        mesh=mesh,
        compiler_params=pltpu.CompilerParams(use_tc_tiling_on_sc=False),
    )
    def gather_kernel(table_hbm, idx_hbm, out_hbm):
        def body(idx_vmem, out_vmem):
            pltpu.sync_copy(
                table_hbm.at[idx_vmem.at[0, pl.ds(0, WINDOW)]],
                out_vmem.at[pl.ds(0, WINDOW)],
            )
            pltpu.sync_copy(
                table_hbm.at[idx_vmem.at[0, pl.ds(WINDOW, WINDOW)]],
                out_vmem.at[pl.ds(WINDOW, WINDOW)],
            )

        pltpu.emit_pipeline(
            body,
            grid=(NUM_IDX // (2 * WINDOW),),
            in_specs=[pl.BlockSpec((1, 2 * WINDOW), index_map=lambda i: (0, i))],
            out_specs=[
                pl.BlockSpec((2 * WINDOW, EMBED_DIM), index_map=lambda i: (i, 0))
            ],
            core_axis_name=("core", "subcore"),
            dimension_semantics=(pltpu.PARALLEL,),
        )(idx_hbm, out_hbm)

    out = gather_kernel(flat_tables, flat_idx)
    out = jnp.transpose(
        out.reshape(NUM_FEATURES, BATCH, EMBED_DIM), (1, 0, 2)
    )
    return out.reshape(BATCH, NUM_FEATURES * EMBED_DIM)


# speedup vs baseline: 1.3274x; 1.3274x over previous
import jax
import jax.numpy as jnp
from jax.experimental import pallas as pl
from jax.experimental.pallas import tpu as pltpu
from jax.experimental.pallas import tpu_sc as plsc

NUM_FEATURES = 26
VOCAB = 100000
EMBED_DIM = 16
BATCH = 16384
NUM_IDX = BATCH * NUM_FEATURES
WINDOW = 128
SUPER_ROWS = NUM_FEATURES * VOCAB // 8
ROWS_PER_F = VOCAB // 8


def _repack_tables(tables):
    tab_t = jnp.transpose(tables, (0, 2, 1)).reshape(
        NUM_FEATURES // 2, 2, EMBED_DIM, VOCAB
    )

    def body(in_ref, out_ref):
        vcc = ROWS_PER_F // 4
        for ff in range(2):
            for c in range(4):
                t = jnp.concatenate(
                    [
                        in_ref[0, ff, :, pl.ds(j * ROWS_PER_F + c * vcc, vcc)]
                        for j in range(8)
                    ],
                    axis=0,
                )
                out_ref[
                    0, pl.ds(ff * ROWS_PER_F + c * vcc, vcc), :
                ] = jnp.transpose(t)

    out = pl.pallas_call(
        body,
        grid=(NUM_FEATURES // 2,),
        in_specs=[
            pl.BlockSpec((1, 2, EMBED_DIM, VOCAB), lambda f: (f, 0, 0, 0))
        ],
        out_specs=pl.BlockSpec((1, 2 * ROWS_PER_F, 128), lambda f: (f, 0, 0)),
        out_shape=jax.ShapeDtypeStruct(
            (NUM_FEATURES // 2, 2 * ROWS_PER_F, 128), jnp.float32
        ),
        compiler_params=pltpu.CompilerParams(
            dimension_semantics=("parallel",),
            vmem_limit_bytes=64 * 1024 * 1024,
        ),
    )(tab_t)
    return out.reshape(SUPER_ROWS, 128)


def kernel(categorical_features, tables):
    flat_tables = _repack_tables(tables).reshape(NUM_FEATURES * VOCAB, EMBED_DIM)
    offs = jnp.arange(NUM_FEATURES, dtype=jnp.int32) * ROWS_PER_F
    v = categorical_features.astype(jnp.int32)
    j, vm = jnp.divmod(v, ROWS_PER_F)
    flat_idx = (8 * (offs[None, :] + vm) + j).reshape(1, NUM_IDX)

    mesh = plsc.VectorSubcoreMesh(core_axis_name="core", subcore_axis_name="subcore")

    @pl.kernel(
        out_type=jax.ShapeDtypeStruct((NUM_IDX, EMBED_DIM), jnp.float32),
        mesh=mesh,
        compiler_params=pltpu.CompilerParams(use_tc_tiling_on_sc=False),
    )
    def gather_kernel(table_hbm, idx_hbm, out_hbm):
        def body(idx_vmem, out_vmem):
            pltpu.sync_copy(
                table_hbm.at[idx_vmem.at[0, pl.ds(0, WINDOW)]],
                out_vmem.at[pl.ds(0, WINDOW)],
            )
            pltpu.sync_copy(
                table_hbm.at[idx_vmem.at[0, pl.ds(WINDOW, WINDOW)]],
                out_vmem.at[pl.ds(WINDOW, WINDOW)],
            )

        pltpu.emit_pipeline(
            body,
            grid=(NUM_IDX // (2 * WINDOW),),
            in_specs=[pl.BlockSpec((1, 2 * WINDOW), index_map=lambda i: (0, i))],
            out_specs=[
                pl.BlockSpec((2 * WINDOW, EMBED_DIM), index_map=lambda i: (i, 0))
            ],
            core_axis_name=("core", "subcore"),
            dimension_semantics=(pltpu.PARALLEL,),
        )(idx_hbm, out_hbm)

    out = gather_kernel(flat_tables, flat_idx)
    return out.reshape(BATCH, NUM_FEATURES * EMBED_DIM)
